# Initial kernel scaffold; baseline (speedup 1.0000x reference)
import jax
import jax.numpy as jnp
from jax.experimental import pallas as pl


def kernel(unary_tensor, binary_tensor, actions, emb_W, emb_b, W_rel, W_root, g_b, c_W1, c_b1, c_W2, c_b2):
    raise NotImplementedError("write your pallas kernel here")


# speedup vs baseline: 405.2585x; 405.2585x over previous
import jax
import jax.numpy as jnp
from jax.experimental import pallas as pl


def _fwd_kernel(x_ref, adj_ref, act_ref, embW_ref, embb_ref, Wrel_ref,
                Wroot_ref, gb_ref, W1_ref, b1_ref, W2_ref, b2_ref, q_ref):
    N = x_ref.shape[1]
    R = adj_ref.shape[1]
    NAG, A = act_ref.shape[1], act_ref.shape[2]

    x = x_ref[0]
    h = jnp.dot(x, embW_ref[...],
                preferred_element_type=jnp.float32) + embb_ref[...]
    acc = jnp.dot(h, Wroot_ref[...],
                  preferred_element_type=jnp.float32) + gb_ref[...]
    ones = jnp.ones((N, 1), jnp.float32)
    for r in range(R):
        Ar = adj_ref[0, r]
        hr = jnp.dot(h, Wrel_ref[r], preferred_element_type=jnp.float32)
        s = jax.lax.dot_general(Ar, hr, (((0,), (0,)), ((), ())),
                                preferred_element_type=jnp.float32)
        c = jax.lax.dot_general(Ar, ones, (((0,), (0,)), ((), ())),
                                preferred_element_type=jnp.float32)
        acc = acc + s / jnp.maximum(c, 1.0)
    out = jnp.maximum(acc, 0.0)
    xg = jnp.max(out, axis=0, keepdims=True)

    iota = jax.lax.broadcasted_iota(jnp.int32, (1, A), 1)
    for a in range(NAG):
        h1 = jnp.dot(xg, W1_ref[a],
                     preferred_element_type=jnp.float32) + b1_ref[a:a + 1]
        h1 = jnp.where(h1 >= 0, h1, 0.01 * h1)
        allq = jnp.dot(h1, W2_ref[a],
                       preferred_element_type=jnp.float32) + b2_ref[a:a + 1]
        act = act_ref[0, a:a + 1, :]
        mx = jnp.max(act, axis=1, keepdims=True)
        first = jnp.min(jnp.where(act >= mx, iota, A),
                        axis=1, keepdims=True)
        q = jnp.sum(jnp.where(iota == first, allq, 0.0),
                    axis=1, keepdims=True)
        q_ref[0, :, a:a + 1] = q


def kernel(unary_tensor, binary_tensor, actions, emb_W, emb_b, W_rel, W_root,
           g_b, c_W1, c_b1, c_W2, c_b2):
    B, N, F = unary_tensor.shape
    R = binary_tensor.shape[3]
    NAG, _, A = actions.shape
    H = emb_W.shape[1]

    adj = jnp.transpose(binary_tensor, (0, 3, 1, 2)).astype(jnp.float32)
    act = jnp.transpose(actions, (1, 0, 2))
    emb_b2 = emb_b.reshape(1, H)
    g_b2 = g_b.reshape(1, H)

    q3 = pl.pallas_call(
        _fwd_kernel,
        grid=(B,),
        in_specs=[
            pl.BlockSpec((1, N, F), lambda b: (b, 0, 0)),
            pl.BlockSpec((1, R, N, N), lambda b: (b, 0, 0, 0)),
            pl.BlockSpec((1, NAG, A), lambda b: (b, 0, 0)),
            pl.BlockSpec((F, H), lambda b: (0, 0)),
            pl.BlockSpec((1, H), lambda b: (0, 0)),
            pl.BlockSpec((R, H, H), lambda b: (0, 0, 0)),
            pl.BlockSpec((H, H), lambda b: (0, 0)),
            pl.BlockSpec((1, H), lambda b: (0, 0)),
            pl.BlockSpec((NAG, H, H), lambda b: (0, 0, 0)),
            pl.BlockSpec((NAG, H), lambda b: (0, 0)),
            pl.BlockSpec((NAG, H, A), lambda b: (0, 0, 0)),
            pl.BlockSpec((NAG, A), lambda b: (0, 0)),
        ],
        out_specs=pl.BlockSpec((1, 1, NAG), lambda b: (b, 0, 0)),
        out_shape=jax.ShapeDtypeStruct((B, 1, NAG), jnp.float32),
    )(unary_tensor.reshape(B, N, F), adj, act, emb_W, emb_b2, W_rel, W_root,
      g_b2, c_W1, c_b1, c_W2, c_b2)

    return q3.reshape(B, NAG).T[:, :, None]


# speedup vs baseline: 1176.5158x; 2.9031x over previous
import jax
import jax.numpy as jnp
from jax.experimental import pallas as pl

_BB = 8


def _fwd_kernel(x_ref, adj_ref, act_ref, embW_ref, embb_ref, Wcat_ref,
                Wroot_ref, gb_ref, W1_ref, b1_ref, W2_ref, b2_ref, q_ref):
    BB, N, F = x_ref.shape
    R = adj_ref.shape[1]
    NAG, A = act_ref.shape[1], act_ref.shape[2]
    H = Wroot_ref.shape[0]

    x = x_ref[...].reshape(BB * N, F)
    h = jnp.dot(x, embW_ref[...],
                preferred_element_type=jnp.float32) + embb_ref[...]
    hr = jnp.dot(h, Wcat_ref[...],
                 preferred_element_type=jnp.float32)
    root = jnp.dot(h, Wroot_ref[...],
                   preferred_element_type=jnp.float32) + gb_ref[...]

    cnt = jnp.sum(adj_ref[...], axis=2)
    rc = 1.0 / jnp.maximum(cnt, 1.0)

    xgs = []
    for b in range(BB):
        acc = root[b * N:(b + 1) * N]
        for r in range(R):
            Ab = adj_ref[b, r] * rc[b, r:r + 1]
            hrb = hr[b * N:(b + 1) * N, r * H:(r + 1) * H]
            acc = acc + jax.lax.dot_general(
                Ab, hrb, (((0,), (0,)), ((), ())),
                preferred_element_type=jnp.float32)
        out = jnp.maximum(acc, 0.0)
        xgs.append(jnp.max(out, axis=0, keepdims=True))
    xg = jnp.concatenate(xgs, axis=0)

    iota = jax.lax.broadcasted_iota(jnp.int32, (BB, A), 1)
    for a in range(NAG):
        h1 = jnp.dot(xg, W1_ref[a],
                     preferred_element_type=jnp.float32) + b1_ref[a:a + 1]
        h1 = jnp.where(h1 >= 0, h1, 0.01 * h1)
        allq = jnp.dot(h1, W2_ref[a],
                       preferred_element_type=jnp.float32) + b2_ref[a:a + 1]
        act = act_ref[:, a, :]
        mx = jnp.max(act, axis=1, keepdims=True)
        first = jnp.min(jnp.where(act >= mx, iota, A),
                        axis=1, keepdims=True)
        q = jnp.sum(jnp.where(iota == first, allq, 0.0),
                    axis=1, keepdims=True)
        q_ref[:, 0, a:a + 1] = q


def kernel(unary_tensor, binary_tensor, actions, emb_W, emb_b, W_rel, W_root,
           g_b, c_W1, c_b1, c_W2, c_b2):
    B, N, F = unary_tensor.shape
    R = binary_tensor.shape[3]
    NAG, _, A = actions.shape
    H = emb_W.shape[1]

    adj = jnp.transpose(binary_tensor, (0, 3, 1, 2)).astype(jnp.float32)
    act = jnp.transpose(actions, (1, 0, 2))
    emb_b2 = emb_b.reshape(1, H)
    g_b2 = g_b.reshape(1, H)
    Wcat = jnp.transpose(W_rel, (1, 0, 2)).reshape(H, R * H)

    q3 = pl.pallas_call(
        _fwd_kernel,
        grid=(B // _BB,),
        in_specs=[
            pl.BlockSpec((_BB, N, F), lambda b: (b, 0, 0)),
            pl.BlockSpec((_BB, R, N, N), lambda b: (b, 0, 0, 0)),
            pl.BlockSpec((_BB, NAG, A), lambda b: (b, 0, 0)),
            pl.BlockSpec((F, H), lambda b: (0, 0)),
            pl.BlockSpec((1, H), lambda b: (0, 0)),
            pl.BlockSpec((H, R * H), lambda b: (0, 0)),
            pl.BlockSpec((H, H), lambda b: (0, 0)),
            pl.BlockSpec((1, H), lambda b: (0, 0)),
            pl.BlockSpec((NAG, H, H), lambda b: (0, 0, 0)),
            pl.BlockSpec((NAG, H), lambda b: (0, 0)),
            pl.BlockSpec((NAG, H, A), lambda b: (0, 0, 0)),
            pl.BlockSpec((NAG, A), lambda b: (0, 0)),
        ],
        out_specs=pl.BlockSpec((_BB, 1, NAG), lambda b: (b, 0, 0)),
        out_shape=jax.ShapeDtypeStruct((B, 1, NAG), jnp.float32),
    )(unary_tensor, adj, act, emb_W, emb_b2, Wcat, W_root,
      g_b2, c_W1, c_b1, c_W2, c_b2)

    return q3.reshape(B, NAG).T[:, :, None]


# speedup vs baseline: 1384.6965x; 1.1769x over previous
import jax
import jax.numpy as jnp
from jax.experimental import pallas as pl

_BB = 8


def _fwd_kernel(x_ref, adj_ref, act_ref, embW_ref, embb_ref, Wcat_ref,
                Wroot_ref, gb_ref, W1_ref, b1_ref, W2_ref, b2_ref, q_ref):
    BB, N, F = x_ref.shape
    R = adj_ref.shape[1]
    NAG, A = act_ref.shape[1], act_ref.shape[2]
    H = Wroot_ref.shape[0]

    x = x_ref[...].reshape(BB * N, F)
    h = jnp.dot(x, embW_ref[...],
                preferred_element_type=jnp.float32) + embb_ref[...]
    hr = jnp.dot(h, Wcat_ref[...],
                 preferred_element_type=jnp.float32)
    root = jnp.dot(h, Wroot_ref[...],
                   preferred_element_type=jnp.float32) + gb_ref[...]

    xgs = []
    for b in range(BB):
        acc = root[b * N:(b + 1) * N]
        for r in range(R):
            Abr = adj_ref[b, r].astype(jnp.float32)
            cnt = jnp.sum(Abr, axis=0, keepdims=True)
            Ab = Abr * (1.0 / jnp.maximum(cnt, 1.0))
            hrb = hr[b * N:(b + 1) * N, r * H:(r + 1) * H]
            acc = acc + jax.lax.dot_general(
                Ab, hrb, (((0,), (0,)), ((), ())),
                preferred_element_type=jnp.float32)
        out = jnp.maximum(acc, 0.0)
        xgs.append(jnp.max(out, axis=0, keepdims=True))
    xg = jnp.concatenate(xgs, axis=0)

    iota = jax.lax.broadcasted_iota(jnp.int32, (BB, A), 1)
    for a in range(NAG):
        h1 = jnp.dot(xg, W1_ref[a],
                     preferred_element_type=jnp.float32) + b1_ref[a:a + 1]
        h1 = jnp.where(h1 >= 0, h1, 0.01 * h1)
        allq = jnp.dot(h1, W2_ref[a],
                       preferred_element_type=jnp.float32) + b2_ref[a:a + 1]
        act = act_ref[:, a, :]
        mx = jnp.max(act, axis=1, keepdims=True)
        first = jnp.min(jnp.where(act >= mx, iota, A),
                        axis=1, keepdims=True)
        q = jnp.sum(jnp.where(iota == first, allq, 0.0),
                    axis=1, keepdims=True)
        q_ref[:, 0, a:a + 1] = q


def kernel(unary_tensor, binary_tensor, actions, emb_W, emb_b, W_rel, W_root,
           g_b, c_W1, c_b1, c_W2, c_b2):
    B, N, F = unary_tensor.shape
    R = binary_tensor.shape[3]
    NAG, _, A = actions.shape
    H = emb_W.shape[1]

    adj = jnp.transpose(binary_tensor, (0, 3, 1, 2)).astype(jnp.int8)
    act = jnp.transpose(actions, (1, 0, 2))
    emb_b2 = emb_b.reshape(1, H)
    g_b2 = g_b.reshape(1, H)
    Wcat = jnp.transpose(W_rel, (1, 0, 2)).reshape(H, R * H)

    q3 = pl.pallas_call(
        _fwd_kernel,
        grid=(B // _BB,),
        in_specs=[
            pl.BlockSpec((_BB, N, F), lambda b: (b, 0, 0)),
            pl.BlockSpec((_BB, R, N, N), lambda b: (b, 0, 0, 0)),
            pl.BlockSpec((_BB, NAG, A), lambda b: (b, 0, 0)),
            pl.BlockSpec((F, H), lambda b: (0, 0)),
            pl.BlockSpec((1, H), lambda b: (0, 0)),
            pl.BlockSpec((H, R * H), lambda b: (0, 0)),
            pl.BlockSpec((H, H), lambda b: (0, 0)),
            pl.BlockSpec((1, H), lambda b: (0, 0)),
            pl.BlockSpec((NAG, H, H), lambda b: (0, 0, 0)),
            pl.BlockSpec((NAG, H), lambda b: (0, 0)),
            pl.BlockSpec((NAG, H, A), lambda b: (0, 0, 0)),
            pl.BlockSpec((NAG, A), lambda b: (0, 0)),
        ],
        out_specs=pl.BlockSpec((_BB, 1, NAG), lambda b: (b, 0, 0)),
        out_shape=jax.ShapeDtypeStruct((B, 1, NAG), jnp.float32),
    )(unary_tensor, adj, act, emb_W, emb_b2, Wcat, W_root,
      g_b2, c_W1, c_b1, c_W2, c_b2)

    return q3.reshape(B, NAG).T[:, :, None]
